# Initial kernel scaffold; baseline (speedup 1.0000x reference)
import jax
import jax.numpy as jnp
from jax.experimental import pallas as pl


def kernel(user_emb, item_emb, v_feat, t_feat, W_v, b_v, W_t, b_t, edge_index):
    raise NotImplementedError("write your pallas kernel here")


# speedup vs baseline: 1.0000x; 1.0000x over previous
import jax
import jax.numpy as jnp
from jax.experimental import pallas as pl

K = 10


def _knn_prop(feat, h, k, layers):
    fn = feat / (jnp.linalg.norm(feat, axis=1, keepdims=True) + 1e-8)
    sim = fn @ fn.T
    vals, idx = jax.lax.top_k(sim, k)
    w = jax.nn.softmax(vals, axis=-1)
    x = h
    for _ in range(layers):
        x = jnp.einsum('ik,ikd->id', w, x[idx])
    return x


def kernel(user_emb, item_emb, v_feat, t_feat, W_v, b_v, W_t, b_t, edge_index):
    num_users = user_emb.shape[0]
    N = num_users + item_emb.shape[0]

    v_h = v_feat @ W_v + b_v
    t_h = t_feat @ W_t + b_t

    v_out = _knn_prop(v_feat, item_emb + v_h, K, 1)
    t_out = _knn_prop(t_feat, item_emb + t_h, K, 1)
    item_h = item_emb + v_out + t_out

    x0 = jnp.concatenate([user_emb, item_h], axis=0)
    u = edge_index[0]
    i = edge_index[1] + num_users
    src = jnp.concatenate([u, i])
    dst = jnp.concatenate([i, u])
    deg = jnp.zeros((N,), x0.dtype).at[dst].add(1.0)
    deg = jnp.maximum(deg, 1.0)
    norm = 1.0 / jnp.sqrt(deg[src] * deg[dst])
    x = x0
    acc = x0
    for _ in range(2):
        msg = x[src] * norm[:, None]
        x = jnp.zeros_like(x).at[dst].add(msg)
        acc = acc + x
    emb = acc / 3.0
    return emb


# speedup vs baseline: 4.3746x; 4.3744x over previous
import dataclasses
import functools

import jax
import jax.numpy as jnp
from jax import lax
from jax.experimental import pallas as pl
from jax.experimental.pallas import tpu as pltpu
from jax.experimental.pallas import tpu_sc as plsc

NU = 5000
NI = 5000
D = 256
KNN = 10
E = 160000

RB = 200
NB = NI // RB
KRB = 128
KNR = 5120
KNB = KNR // KRB

NSUB = 16
NPAD = 5120
RPW = NPAD // NSUB
EC = 64
ECD = 128


def _prep_body(vf_ref, tf_ref, ie_ref, wv_ref, bv_ref, wt_ref, bt_ref,
               fnv_ref, fnt_ref, hv_ref, ht_ref):
    vf = vf_ref[...]
    tf = tf_ref[...]
    ie = ie_ref[...]
    nv = jnp.sqrt(jnp.sum(vf * vf, axis=1, keepdims=True)) + 1e-8
    fnv_ref[...] = vf / nv
    nt = jnp.sqrt(jnp.sum(tf * tf, axis=1, keepdims=True)) + 1e-8
    fnt_ref[...] = tf / nt
    hv_ref[...] = ie + jnp.dot(vf, wv_ref[...],
                               preferred_element_type=jnp.float32) + bv_ref[...]
    ht_ref[...] = ie + jnp.dot(tf, wt_ref[...],
                               preferred_element_type=jnp.float32) + bt_ref[...]


def _prep(v_feat, t_feat, item_emb, W_v, b_v, W_t, b_t):
    return pl.pallas_call(
        _prep_body,
        grid=(NB,),
        in_specs=[
            pl.BlockSpec((RB, 512), lambda i: (i, 0)),
            pl.BlockSpec((RB, 384), lambda i: (i, 0)),
            pl.BlockSpec((RB, D), lambda i: (i, 0)),
            pl.BlockSpec((512, D), lambda i: (0, 0)),
            pl.BlockSpec((1, D), lambda i: (0, 0)),
            pl.BlockSpec((384, D), lambda i: (0, 0)),
            pl.BlockSpec((1, D), lambda i: (0, 0)),
        ],
        out_specs=[
            pl.BlockSpec((RB, 512), lambda i: (i, 0)),
            pl.BlockSpec((RB, 384), lambda i: (i, 0)),
            pl.BlockSpec((RB, D), lambda i: (i, 0)),
            pl.BlockSpec((RB, D), lambda i: (i, 0)),
        ],
        out_shape=[
            jax.ShapeDtypeStruct((NI, 512), jnp.float32),
            jax.ShapeDtypeStruct((NI, 384), jnp.float32),
            jax.ShapeDtypeStruct((NI, D), jnp.float32),
            jax.ShapeDtypeStruct((NI, D), jnp.float32),
        ],
    )(v_feat, t_feat, item_emb, W_v, b_v.reshape(1, D), W_t, b_t.reshape(1, D))


def _knn_body(fb_ref, fT_ref, h_ref, o_ref):
    fb = fb_ref[...]
    scores = jnp.dot(fb, fT_ref[...], preferred_element_type=jnp.float32)
    m0 = jnp.max(scores, axis=1, keepdims=True)
    sel = jnp.zeros(scores.shape, jnp.float32)
    denom = jnp.zeros((scores.shape[0], 1), jnp.float32)
    for t in range(KNN):
        m = m0 if t == 0 else jnp.max(scores, axis=1, keepdims=True)
        w = jnp.exp(m - m0)
        hit = scores >= m
        sel = sel + w * hit.astype(jnp.float32)
        denom = denom + w
        scores = jnp.where(hit, -1e30, scores)
    out = jnp.dot(sel, h_ref[...], preferred_element_type=jnp.float32)
    o_ref[...] = out / denom


def _knn(fn, fnT, h, kd):
    fn_pad = jnp.concatenate(
        [fn, jnp.zeros((KNR - NI, kd), jnp.float32)], axis=0)
    out = pl.pallas_call(
        _knn_body,
        grid=(KNB,),
        in_specs=[
            pl.BlockSpec((KRB, kd), lambda i: (i, 0)),
            pl.BlockSpec((kd, NI), lambda i: (0, 0)),
            pl.BlockSpec((NI, D), lambda i: (0, 0)),
        ],
        out_specs=pl.BlockSpec((KRB, D), lambda i: (i, 0)),
        out_shape=jax.ShapeDtypeStruct((KNR, D), jnp.float32),
    )(fn_pad, fnT, h)
    return out[:NI]


def _worker_bounds(lo_hbm, hi_hbm, bnd_v, cid, sid):
    iota = lax.iota(jnp.int32, NSUB)
    pltpu.sync_copy(lo_hbm.at[cid], bnd_v)
    lo = jnp.sum(jnp.where(iota == sid, bnd_v[...], 0))
    pltpu.sync_copy(hi_hbm.at[cid], bnd_v)
    hi = jnp.sum(jnp.where(iota == sid, bnd_v[...], 0))
    return lo, hi


def _deg_body(dsti_hbm, dstu_hbm, lo_hbm, hi_hbm, out_hbm,
              didx_v, acc_v, bnd_v, sem):
    cid = lax.axis_index("c")
    sid = lax.axis_index("s")
    lo, hi = _worker_bounds(lo_hbm, hi_hbm, bnd_v, cid, sid)
    rbase = sid * RPW

    @pl.loop(0, RPW)
    def _(i):
        acc_v[pl.ds(i * 16, 16)] = jnp.zeros((16,), jnp.float32)

    ones = jnp.ones((16,), jnp.float32)

    def scan(dst_hbm):
        @pl.loop(lo // ECD, (hi + ECD - 1) // ECD)
        def _(t):
            base = t * ECD
            pltpu.sync_copy(dst_hbm.at[pl.ds(base, ECD)], didx_v)
            for jv in range(ECD // 16):
                dvec = didx_v[pl.ds(jv * 16, 16)] - rbase
                for j in range(16):
                    g = base + jv * 16 + j
                    dloc = dvec[j]

                    @pl.when(jnp.logical_and(g >= lo, g < hi))
                    def _():
                        plsc.addupdate(acc_v.at[pl.ds(dloc * 16, 16)], ones)

    @pl.when(cid == 0)
    def _():
        scan(dsti_hbm)

    @pl.when(cid == 1)
    def _():
        scan(dstu_hbm)

    pltpu.sync_copy(acc_v, out_hbm.at[cid].at[pl.ds(sid * RPW * 16, RPW * 16)])


def _seg_body(yu_hbm, yi_hbm, srcu_hbm, srci_hbm, dsti_hbm, dstu_hbm,
              lo_hbm, hi_hbm, out_hbm, sidx_v, didx_v, rows_v, acc_v, bnd_v,
              sem):
    cid = lax.axis_index("c")
    sid = lax.axis_index("s")
    lo, hi = _worker_bounds(lo_hbm, hi_hbm, bnd_v, cid, sid)
    rbase = sid * RPW

    @pl.loop(0, RPW * D // 16)
    def _(i):
        acc_v[pl.ds(i * 16, 16)] = jnp.zeros((16,), jnp.float32)

    def scan(tbl_hbm, src_hbm, dst_hbm):
        @pl.loop(lo // EC, (hi + EC - 1) // EC)
        def _(t):
            base = t * EC
            pltpu.sync_copy(src_hbm.at[pl.ds(base, EC)], sidx_v)
            pltpu.sync_copy(dst_hbm.at[pl.ds(base, EC)], didx_v)
            pltpu.async_copy(tbl_hbm.at[sidx_v], rows_v, sem).wait()
            for jv in range(EC // 16):
                dvec = didx_v[pl.ds(jv * 16, 16)] - rbase
                for j in range(16):
                    g = base + jv * 16 + j
                    dloc = dvec[j]

                    @pl.when(jnp.logical_and(g >= lo, g < hi))
                    def _():
                        off = dloc * D
                        for kk in range(D // 16):
                            val = rows_v[jv * 16 + j, pl.ds(kk * 16, 16)]
                            plsc.addupdate(
                                acc_v.at[pl.ds(off + kk * 16, 16)], val)

    @pl.when(cid == 0)
    def _():
        scan(yu_hbm, srcu_hbm, dsti_hbm)

    @pl.when(cid == 1)
    def _():
        scan(yi_hbm, srci_hbm, dstu_hbm)

    pltpu.sync_copy(acc_v, out_hbm.at[cid].at[pl.ds(sid * RPW * D, RPW * D)])


@functools.cache
def _sc_kernels():
    mesh = plsc.VectorSubcoreMesh(core_axis_name="c", subcore_axis_name="s")
    cp = pltpu.CompilerParams()
    if "needs_layout_passes" in pltpu.CompilerParams.__dataclass_fields__:
        cp = dataclasses.replace(cp, needs_layout_passes=False)

    deg = functools.partial(
        pl.kernel, mesh=mesh, compiler_params=cp,
        out_type=jax.ShapeDtypeStruct((2, NPAD * 16), jnp.float32),
        scratch_types=[
            pltpu.VMEM((ECD,), jnp.int32),
            pltpu.VMEM((RPW * 16,), jnp.float32),
            pltpu.VMEM((NSUB,), jnp.int32),
            pltpu.SemaphoreType.DMA,
        ],
    )(_deg_body)
    seg = functools.partial(
        pl.kernel, mesh=mesh, compiler_params=cp,
        out_type=jax.ShapeDtypeStruct((2, NPAD * D), jnp.float32),
        scratch_types=[
            pltpu.VMEM((EC,), jnp.int32),
            pltpu.VMEM((EC,), jnp.int32),
            pltpu.VMEM((EC, D), jnp.float32),
            pltpu.VMEM((RPW * D,), jnp.float32),
            pltpu.VMEM((NSUB,), jnp.int32),
            pltpu.SemaphoreType.DMA,
        ],
    )(_seg_body)
    return deg, seg


def kernel(user_emb, item_emb, v_feat, t_feat, W_v, b_v, W_t, b_t, edge_index):
    fnv, fnt, hv, ht = _prep(v_feat, t_feat, item_emb, W_v, b_v, W_t, b_t)
    v_out = _knn(fnv, fnv.T, hv, 512)
    t_out = _knn(fnt, fnt.T, ht, 384)
    item_h = item_emb + v_out + t_out

    uids = edge_index[0].astype(jnp.int32)
    iids = edge_index[1].astype(jnp.int32)
    dsti, srcu = lax.sort((iids, uids), num_keys=1)
    dstu, srci = lax.sort((uids, iids), num_keys=1)
    marks = jnp.arange(NSUB + 1, dtype=jnp.int32) * RPW
    bnd_i = jnp.searchsorted(dsti, marks).astype(jnp.int32)
    bnd_u = jnp.searchsorted(dstu, marks).astype(jnp.int32)
    lo = jnp.stack([bnd_i[:NSUB], bnd_u[:NSUB]])
    hi = jnp.stack([bnd_i[1:], bnd_u[1:]])

    deg_kernel, seg_kernel = _sc_kernels()
    cnt = deg_kernel(dsti, dstu, lo, hi)
    deg_i = cnt[0].reshape(NPAD, 16)[:NI, 0]
    deg_u = cnt[1].reshape(NPAD, 16)[:NU, 0]
    a = 1.0 / jnp.sqrt(jnp.maximum(deg_u, 1.0))
    b = 1.0 / jnp.sqrt(jnp.maximum(deg_i, 1.0))

    s1 = seg_kernel(user_emb * a[:, None], item_h * b[:, None],
                    srcu, srci, dsti, dstu, lo, hi)
    xi1 = s1[0].reshape(NPAD, D)[:NI] * b[:, None]
    xu1 = s1[1].reshape(NPAD, D)[:NU] * a[:, None]

    s2 = seg_kernel(xu1 * a[:, None], xi1 * b[:, None],
                    srcu, srci, dsti, dstu, lo, hi)
    xi2 = s2[0].reshape(NPAD, D)[:NI] * b[:, None]
    xu2 = s2[1].reshape(NPAD, D)[:NU] * a[:, None]

    emb_u = (user_emb + xu1 + xu2) * (1.0 / 3.0)
    emb_i = (item_h + xi1 + xi2) * (1.0 / 3.0)
    return jnp.concatenate([emb_u, emb_i], axis=0)


# speedup vs baseline: 4.8208x; 1.1020x over previous
import dataclasses
import functools

import jax
import jax.numpy as jnp
from jax import lax
from jax.experimental import pallas as pl
from jax.experimental.pallas import tpu as pltpu
from jax.experimental.pallas import tpu_sc as plsc

NU = 5000
NI = 5000
D = 256
KNN = 10
E = 160000

RB = 200
NB = NI // RB
KRB = 128
KNR = 5120
KNB = KNR // KRB

NSUB = 16
NPAD = 5120
RPW = NPAD // NSUB
EC = 32
SCC = 1024
NCPS = SCC // EC
EPAD = -(-E // SCC) * SCC
ECD = 128


def _prep_body(vf_ref, tf_ref, ie_ref, wv_ref, bv_ref, wt_ref, bt_ref,
               fnv_ref, fnt_ref, hv_ref, ht_ref):
    vf = vf_ref[...]
    tf = tf_ref[...]
    ie = ie_ref[...]
    nv = jnp.sqrt(jnp.sum(vf * vf, axis=1, keepdims=True)) + 1e-8
    fnv_ref[...] = vf / nv
    nt = jnp.sqrt(jnp.sum(tf * tf, axis=1, keepdims=True)) + 1e-8
    fnt_ref[...] = tf / nt
    hv_ref[...] = ie + jnp.dot(vf, wv_ref[...],
                               preferred_element_type=jnp.float32) + bv_ref[...]
    ht_ref[...] = ie + jnp.dot(tf, wt_ref[...],
                               preferred_element_type=jnp.float32) + bt_ref[...]


def _prep(v_feat, t_feat, item_emb, W_v, b_v, W_t, b_t):
    return pl.pallas_call(
        _prep_body,
        grid=(NB,),
        in_specs=[
            pl.BlockSpec((RB, 512), lambda i: (i, 0)),
            pl.BlockSpec((RB, 384), lambda i: (i, 0)),
            pl.BlockSpec((RB, D), lambda i: (i, 0)),
            pl.BlockSpec((512, D), lambda i: (0, 0)),
            pl.BlockSpec((1, D), lambda i: (0, 0)),
            pl.BlockSpec((384, D), lambda i: (0, 0)),
            pl.BlockSpec((1, D), lambda i: (0, 0)),
        ],
        out_specs=[
            pl.BlockSpec((RB, 512), lambda i: (i, 0)),
            pl.BlockSpec((RB, 384), lambda i: (i, 0)),
            pl.BlockSpec((RB, D), lambda i: (i, 0)),
            pl.BlockSpec((RB, D), lambda i: (i, 0)),
        ],
        out_shape=[
            jax.ShapeDtypeStruct((NI, 512), jnp.float32),
            jax.ShapeDtypeStruct((NI, 384), jnp.float32),
            jax.ShapeDtypeStruct((NI, D), jnp.float32),
            jax.ShapeDtypeStruct((NI, D), jnp.float32),
        ],
    )(v_feat, t_feat, item_emb, W_v, b_v.reshape(1, D), W_t, b_t.reshape(1, D))


def _knn_body(fb_ref, fT_ref, h_ref, o_ref):
    fb = fb_ref[...]
    scores = jnp.dot(fb, fT_ref[...], preferred_element_type=jnp.float32)
    m0 = jnp.max(scores, axis=1, keepdims=True)
    sel = jnp.zeros(scores.shape, jnp.float32)
    denom = jnp.zeros((scores.shape[0], 1), jnp.float32)
    for t in range(KNN):
        m = m0 if t == 0 else jnp.max(scores, axis=1, keepdims=True)
        w = jnp.exp(m - m0)
        hit = scores >= m
        sel = sel + w * hit.astype(jnp.float32)
        denom = denom + w
        scores = jnp.where(hit, -1e30, scores)
    out = jnp.dot(sel, h_ref[...], preferred_element_type=jnp.float32)
    o_ref[...] = out / denom


def _knn(fn, fnT, h, kd):
    fn_pad = jnp.concatenate(
        [fn, jnp.zeros((KNR - NI, kd), jnp.float32)], axis=0)
    out = pl.pallas_call(
        _knn_body,
        grid=(KNB,),
        in_specs=[
            pl.BlockSpec((KRB, kd), lambda i: (i, 0)),
            pl.BlockSpec((kd, NI), lambda i: (0, 0)),
            pl.BlockSpec((NI, D), lambda i: (0, 0)),
        ],
        out_specs=pl.BlockSpec((KRB, D), lambda i: (i, 0)),
        out_shape=jax.ShapeDtypeStruct((KNR, D), jnp.float32),
    )(fn_pad, fnT, h)
    return out[:NI]


def _worker_bounds(lo_hbm, hi_hbm, bnd_v, cid, sid):
    iota = lax.iota(jnp.int32, NSUB)
    pltpu.sync_copy(lo_hbm.at[cid], bnd_v)
    lo = jnp.sum(jnp.where(iota == sid, bnd_v[...], 0))
    pltpu.sync_copy(hi_hbm.at[cid], bnd_v)
    hi = jnp.sum(jnp.where(iota == sid, bnd_v[...], 0))
    return lo, hi


def _deg_body(dsti_hbm, dstu_hbm, lo_hbm, hi_hbm, out_hbm,
              didx_v, acc_v, bnd_v, sem):
    cid = lax.axis_index("c")
    sid = lax.axis_index("s")
    lo, hi = _worker_bounds(lo_hbm, hi_hbm, bnd_v, cid, sid)
    rbase = sid * RPW

    @pl.loop(0, RPW)
    def _(i):
        acc_v[pl.ds(i * 16, 16)] = jnp.zeros((16,), jnp.float32)

    ones = jnp.ones((16,), jnp.float32)

    def scan(dst_hbm):
        @pl.loop(lo // ECD, (hi + ECD - 1) // ECD)
        def _(t):
            base = t * ECD
            pltpu.sync_copy(dst_hbm.at[pl.ds(base, ECD)], didx_v)
            for jv in range(ECD // 16):
                dvec = didx_v[pl.ds(jv * 16, 16)] - rbase
                for j in range(16):
                    g = base + jv * 16 + j
                    dloc = dvec[j]

                    @pl.when(jnp.logical_and(g >= lo, g < hi))
                    def _():
                        plsc.addupdate(acc_v.at[pl.ds(dloc * 16, 16)], ones)

    @pl.when(cid == 0)
    def _():
        scan(dsti_hbm)

    @pl.when(cid == 1)
    def _():
        scan(dstu_hbm)

    pltpu.sync_copy(acc_v, out_hbm.at[cid].at[pl.ds(sid * RPW * 16, RPW * 16)])


def _seg_body(yu_hbm, yi_hbm, srcu_hbm, srci_hbm, dsti_hbm, dstu_hbm,
              lo_hbm, hi_hbm, out_hbm, sidx_v, didx_v, rows0_v, rows1_v,
              acc_v, bnd_v, sem0, sem1):
    cid = lax.axis_index("c")
    sid = lax.axis_index("s")
    lo, hi = _worker_bounds(lo_hbm, hi_hbm, bnd_v, cid, sid)
    rbase = sid * RPW

    @pl.loop(0, RPW * D // 16)
    def _(i):
        acc_v[pl.ds(i * 16, 16)] = jnp.zeros((16,), jnp.float32)

    def scan(tbl_hbm, src_hbm, dst_hbm):
        rows = (rows0_v, rows1_v)
        sems = (sem0, sem1)

        def issue(k, b):
            pltpu.async_copy(tbl_hbm.at[sidx_v.at[pl.ds(k * EC, EC)]],
                             rows[b], sems[b])

        def process(k, b, base):
            for jv in range(EC // 16):
                dvec = didx_v[pl.ds(k * EC + jv * 16, 16)] - rbase
                for j in range(16):
                    g = base + k * EC + jv * 16 + j
                    dloc = dvec[j]

                    @pl.when(jnp.logical_and(g >= lo, g < hi))
                    def _():
                        off = dloc * D
                        for kk in range(D // 16):
                            val = rows[b][jv * 16 + j, pl.ds(kk * 16, 16)]
                            plsc.addupdate(
                                acc_v.at[pl.ds(off + kk * 16, 16)], val)

        @pl.loop(lo // SCC, (hi + SCC - 1) // SCC)
        def _(sb):
            base = sb * SCC
            pltpu.sync_copy(src_hbm.at[pl.ds(base, SCC)], sidx_v)
            pltpu.sync_copy(dst_hbm.at[pl.ds(base, SCC)], didx_v)
            issue(0, 0)
            issue(1, 1)

            @pl.loop(0, NCPS, step=2)
            def _(k):
                pltpu.make_async_copy(
                    tbl_hbm.at[sidx_v.at[pl.ds(0, EC)]], rows[0],
                    sems[0]).wait()
                process(k, 0, base)

                @pl.when(k + 2 < NCPS)
                def _():
                    issue(k + 2, 0)

                pltpu.make_async_copy(
                    tbl_hbm.at[sidx_v.at[pl.ds(0, EC)]], rows[1],
                    sems[1]).wait()
                process(k + 1, 1, base)

                @pl.when(k + 3 < NCPS)
                def _():
                    issue(k + 3, 1)

    @pl.when(cid == 0)
    def _():
        scan(yu_hbm, srcu_hbm, dsti_hbm)

    @pl.when(cid == 1)
    def _():
        scan(yi_hbm, srci_hbm, dstu_hbm)

    pltpu.sync_copy(acc_v, out_hbm.at[cid].at[pl.ds(sid * RPW * D, RPW * D)])


@functools.cache
def _sc_kernels():
    mesh = plsc.VectorSubcoreMesh(core_axis_name="c", subcore_axis_name="s")
    cp = pltpu.CompilerParams()
    if "needs_layout_passes" in pltpu.CompilerParams.__dataclass_fields__:
        cp = dataclasses.replace(cp, needs_layout_passes=False)

    deg = functools.partial(
        pl.kernel, mesh=mesh, compiler_params=cp,
        out_type=jax.ShapeDtypeStruct((2, NPAD * 16), jnp.float32),
        scratch_types=[
            pltpu.VMEM((ECD,), jnp.int32),
            pltpu.VMEM((RPW * 16,), jnp.float32),
            pltpu.VMEM((NSUB,), jnp.int32),
            pltpu.SemaphoreType.DMA,
        ],
    )(_deg_body)
    seg = functools.partial(
        pl.kernel, mesh=mesh, compiler_params=cp,
        out_type=jax.ShapeDtypeStruct((2, NPAD * D), jnp.float32),
        scratch_types=[
            pltpu.VMEM((SCC,), jnp.int32),
            pltpu.VMEM((SCC,), jnp.int32),
            pltpu.VMEM((EC, D), jnp.float32),
            pltpu.VMEM((EC, D), jnp.float32),
            pltpu.VMEM((RPW * D,), jnp.float32),
            pltpu.VMEM((NSUB,), jnp.int32),
            pltpu.SemaphoreType.DMA,
            pltpu.SemaphoreType.DMA,
        ],
    )(_seg_body)
    return deg, seg


def kernel(user_emb, item_emb, v_feat, t_feat, W_v, b_v, W_t, b_t, edge_index):
    fnv, fnt, hv, ht = _prep(v_feat, t_feat, item_emb, W_v, b_v, W_t, b_t)
    v_out = _knn(fnv, fnv.T, hv, 512)
    t_out = _knn(fnt, fnt.T, ht, 384)
    item_h = item_emb + v_out + t_out

    uids = edge_index[0].astype(jnp.int32)
    iids = edge_index[1].astype(jnp.int32)
    dsti, srcu = lax.sort((iids, uids), num_keys=1)
    dstu, srci = lax.sort((uids, iids), num_keys=1)
    zpad = jnp.zeros((EPAD - E,), jnp.int32)
    dsti_p = jnp.concatenate([dsti, zpad])
    srcu_p = jnp.concatenate([srcu, zpad])
    dstu_p = jnp.concatenate([dstu, zpad])
    srci_p = jnp.concatenate([srci, zpad])
    marks = jnp.arange(NSUB + 1, dtype=jnp.int32) * RPW
    bnd_i = jnp.searchsorted(dsti, marks).astype(jnp.int32)
    bnd_u = jnp.searchsorted(dstu, marks).astype(jnp.int32)
    lo = jnp.stack([bnd_i[:NSUB], bnd_u[:NSUB]])
    hi = jnp.stack([bnd_i[1:], bnd_u[1:]])

    deg_kernel, seg_kernel = _sc_kernels()
    cnt = deg_kernel(dsti, dstu, lo, hi)
    deg_i = cnt[0].reshape(NPAD, 16)[:NI, 0]
    deg_u = cnt[1].reshape(NPAD, 16)[:NU, 0]
    a = 1.0 / jnp.sqrt(jnp.maximum(deg_u, 1.0))
    b = 1.0 / jnp.sqrt(jnp.maximum(deg_i, 1.0))

    s1 = seg_kernel(user_emb * a[:, None], item_h * b[:, None],
                    srcu_p, srci_p, dsti_p, dstu_p, lo, hi)
    xi1 = s1[0].reshape(NPAD, D)[:NI] * b[:, None]
    xu1 = s1[1].reshape(NPAD, D)[:NU] * a[:, None]

    s2 = seg_kernel(xu1 * a[:, None], xi1 * b[:, None],
                    srcu_p, srci_p, dsti_p, dstu_p, lo, hi)
    xi2 = s2[0].reshape(NPAD, D)[:NI] * b[:, None]
    xu2 = s2[1].reshape(NPAD, D)[:NU] * a[:, None]

    emb_u = (user_emb + xu1 + xu2) * (1.0 / 3.0)
    emb_i = (item_h + xi1 + xi2) * (1.0 / 3.0)
    return jnp.concatenate([emb_u, emb_i], axis=0)


# speedup vs baseline: 5.0426x; 1.0460x over previous
import dataclasses
import functools

import jax
import jax.numpy as jnp
from jax import lax
from jax.experimental import pallas as pl
from jax.experimental.pallas import tpu as pltpu
from jax.experimental.pallas import tpu_sc as plsc

NU = 5000
NI = 5000
D = 256
KNN = 10
E = 160000

RB = 200
NB = NI // RB
KRB = 128
KNR = 5120
KNB = KNR // KRB

NSUB = 16
NPAD = 5120
RPW = NPAD // NSUB
EC = 32
SCC = 1024
NCPS = SCC // EC
EPAD = -(-E // SCC) * SCC
ECD = 128
PACK = 8192
SHIFT = 13


def _prep_body(vf_ref, tf_ref, ie_ref, wv_ref, bv_ref, wt_ref, bt_ref,
               fnv_ref, fnt_ref, hv_ref, ht_ref):
    vf = vf_ref[...]
    tf = tf_ref[...]
    ie = ie_ref[...]
    nv = jnp.sqrt(jnp.sum(vf * vf, axis=1, keepdims=True)) + 1e-8
    fnv_ref[...] = vf / nv
    nt = jnp.sqrt(jnp.sum(tf * tf, axis=1, keepdims=True)) + 1e-8
    fnt_ref[...] = tf / nt
    hv_ref[...] = ie + jnp.dot(vf, wv_ref[...],
                               preferred_element_type=jnp.float32) + bv_ref[...]
    ht_ref[...] = ie + jnp.dot(tf, wt_ref[...],
                               preferred_element_type=jnp.float32) + bt_ref[...]


def _prep(v_feat, t_feat, item_emb, W_v, b_v, W_t, b_t):
    return pl.pallas_call(
        _prep_body,
        grid=(NB,),
        in_specs=[
            pl.BlockSpec((RB, 512), lambda i: (i, 0)),
            pl.BlockSpec((RB, 384), lambda i: (i, 0)),
            pl.BlockSpec((RB, D), lambda i: (i, 0)),
            pl.BlockSpec((512, D), lambda i: (0, 0)),
            pl.BlockSpec((1, D), lambda i: (0, 0)),
            pl.BlockSpec((384, D), lambda i: (0, 0)),
            pl.BlockSpec((1, D), lambda i: (0, 0)),
        ],
        out_specs=[
            pl.BlockSpec((RB, 512), lambda i: (i, 0)),
            pl.BlockSpec((RB, 384), lambda i: (i, 0)),
            pl.BlockSpec((RB, D), lambda i: (i, 0)),
            pl.BlockSpec((RB, D), lambda i: (i, 0)),
        ],
        out_shape=[
            jax.ShapeDtypeStruct((NI, 512), jnp.float32),
            jax.ShapeDtypeStruct((NI, 384), jnp.float32),
            jax.ShapeDtypeStruct((NI, D), jnp.float32),
            jax.ShapeDtypeStruct((NI, D), jnp.float32),
        ],
    )(v_feat, t_feat, item_emb, W_v, b_v.reshape(1, D), W_t, b_t.reshape(1, D))


def _knn_body(fb_ref, fT_ref, h_ref, o_ref):
    fb = fb_ref[...]
    scores = jnp.dot(fb, fT_ref[...], preferred_element_type=jnp.float32)
    m0 = jnp.max(scores, axis=1, keepdims=True)
    sel = jnp.zeros(scores.shape, jnp.float32)
    denom = jnp.zeros((scores.shape[0], 1), jnp.float32)
    for t in range(KNN):
        m = m0 if t == 0 else jnp.max(scores, axis=1, keepdims=True)
        w = jnp.exp(m - m0)
        hit = scores >= m
        sel = sel + w * hit.astype(jnp.float32)
        denom = denom + w
        scores = jnp.where(hit, -1e30, scores)
    out = jnp.dot(sel, h_ref[...], preferred_element_type=jnp.float32)
    o_ref[...] = out / denom


def _knn(fn, fnT, h, kd):
    fn_pad = jnp.concatenate(
        [fn, jnp.zeros((KNR - NI, kd), jnp.float32)], axis=0)
    out = pl.pallas_call(
        _knn_body,
        grid=(KNB,),
        in_specs=[
            pl.BlockSpec((KRB, kd), lambda i: (i, 0)),
            pl.BlockSpec((kd, NI), lambda i: (0, 0)),
            pl.BlockSpec((NI, D), lambda i: (0, 0)),
        ],
        out_specs=pl.BlockSpec((KRB, D), lambda i: (i, 0)),
        out_shape=jax.ShapeDtypeStruct((KNR, D), jnp.float32),
    )(fn_pad, fnT, h)
    return out[:NI]


def _worker_bounds(lo_hbm, hi_hbm, bnd_v, cid, sid):
    iota = lax.iota(jnp.int32, NSUB)
    pltpu.sync_copy(lo_hbm.at[cid], bnd_v)
    lo = jnp.sum(jnp.where(iota == sid, bnd_v[...], 0))
    pltpu.sync_copy(hi_hbm.at[cid], bnd_v)
    hi = jnp.sum(jnp.where(iota == sid, bnd_v[...], 0))
    return lo, hi


def _deg_body(keyi_hbm, keyu_hbm, lo_hbm, hi_hbm, out_hbm,
              kidx_v, acc_v, bnd_v, sem):
    cid = lax.axis_index("c")
    sid = lax.axis_index("s")
    lo, hi = _worker_bounds(lo_hbm, hi_hbm, bnd_v, cid, sid)
    rbase = sid * RPW
    iota = lax.iota(jnp.int32, 16)

    @pl.loop(0, RPW + 1)
    def _(i):
        acc_v[pl.ds(i * 16, 16)] = jnp.zeros((16,), jnp.float32)

    ones = jnp.ones((16,), jnp.float32)

    def scan(key_hbm):
        @pl.loop(lo // ECD, (hi + ECD - 1) // ECD)
        def _(t):
            base = t * ECD
            pltpu.sync_copy(key_hbm.at[pl.ds(base, ECD)], kidx_v)
            for jv in range(ECD // 16):
                kvec = kidx_v[pl.ds(jv * 16, 16)]
                dvec = lax.shift_right_logical(kvec, SHIFT)
                gvec = base + jv * 16 + iota
                valid = jnp.logical_and(gvec >= lo, gvec < hi)
                offv = jnp.where(valid, dvec - rbase, RPW) * 16
                for j in range(16):
                    plsc.addupdate(acc_v.at[pl.ds(offv[j], 16)], ones)

    @pl.when(cid == 0)
    def _():
        scan(keyi_hbm)

    @pl.when(cid == 1)
    def _():
        scan(keyu_hbm)

    pltpu.sync_copy(acc_v.at[pl.ds(0, RPW * 16)],
                    out_hbm.at[cid].at[pl.ds(sid * RPW * 16, RPW * 16)])


def _seg_body(yu_hbm, yi_hbm, keyi_hbm, keyu_hbm,
              lo_hbm, hi_hbm, out_hbm, sidx_v, kidx_v, rows0_v, rows1_v,
              acc_v, bnd_v, sem0, sem1):
    cid = lax.axis_index("c")
    sid = lax.axis_index("s")
    lo, hi = _worker_bounds(lo_hbm, hi_hbm, bnd_v, cid, sid)
    rbase = sid * RPW
    iota = lax.iota(jnp.int32, 16)

    @pl.loop(0, RPW * (D // 16) + D // 16)
    def _(i):
        acc_v[pl.ds(i * 16, 16)] = jnp.zeros((16,), jnp.float32)

    def scan(tbl_hbm, key_hbm):
        rows = (rows0_v, rows1_v)
        sems = (sem0, sem1)

        def issue(k, b):
            pltpu.async_copy(tbl_hbm.at[sidx_v.at[pl.ds(k * EC, EC)]],
                             rows[b], sems[b])

        def process(k, b, base):
            for jv in range(EC // 16):
                kvec = kidx_v[pl.ds(k * EC + jv * 16, 16)]
                dvec = lax.shift_right_logical(kvec, SHIFT)
                gvec = base + k * EC + jv * 16 + iota
                valid = jnp.logical_and(gvec >= lo, gvec < hi)
                offv = jnp.where(valid, dvec - rbase, RPW) * D
                for j in range(16):
                    off = offv[j]
                    for kk in range(D // 16):
                        val = rows[b][jv * 16 + j, pl.ds(kk * 16, 16)]
                        plsc.addupdate(
                            acc_v.at[pl.ds(off + kk * 16, 16)], val)

        @pl.loop(lo // SCC, (hi + SCC - 1) // SCC)
        def _(sb):
            base = sb * SCC
            pltpu.sync_copy(key_hbm.at[pl.ds(base, SCC)], kidx_v)

            @pl.loop(0, SCC // 16)
            def _(i):
                sidx_v[pl.ds(i * 16, 16)] = jnp.bitwise_and(
                    kidx_v[pl.ds(i * 16, 16)], PACK - 1)

            issue(0, 0)
            issue(1, 1)

            @pl.loop(0, NCPS, step=2)
            def _(k):
                pltpu.make_async_copy(
                    tbl_hbm.at[sidx_v.at[pl.ds(0, EC)]], rows[0],
                    sems[0]).wait()
                process(k, 0, base)

                @pl.when(k + 2 < NCPS)
                def _():
                    issue(k + 2, 0)

                pltpu.make_async_copy(
                    tbl_hbm.at[sidx_v.at[pl.ds(0, EC)]], rows[1],
                    sems[1]).wait()
                process(k + 1, 1, base)

                @pl.when(k + 3 < NCPS)
                def _():
                    issue(k + 3, 1)

    @pl.when(cid == 0)
    def _():
        scan(yu_hbm, keyi_hbm)

    @pl.when(cid == 1)
    def _():
        scan(yi_hbm, keyu_hbm)

    pltpu.sync_copy(acc_v.at[pl.ds(0, RPW * D)],
                    out_hbm.at[cid].at[pl.ds(sid * RPW * D, RPW * D)])


@functools.cache
def _sc_kernels():
    mesh = plsc.VectorSubcoreMesh(core_axis_name="c", subcore_axis_name="s")
    cp = pltpu.CompilerParams()
    if "needs_layout_passes" in pltpu.CompilerParams.__dataclass_fields__:
        cp = dataclasses.replace(cp, needs_layout_passes=False)

    deg = functools.partial(
        pl.kernel, mesh=mesh, compiler_params=cp,
        out_type=jax.ShapeDtypeStruct((2, NPAD * 16), jnp.float32),
        scratch_types=[
            pltpu.VMEM((ECD,), jnp.int32),
            pltpu.VMEM((RPW * 16 + 16,), jnp.float32),
            pltpu.VMEM((NSUB,), jnp.int32),
            pltpu.SemaphoreType.DMA,
        ],
    )(_deg_body)
    seg = functools.partial(
        pl.kernel, mesh=mesh, compiler_params=cp,
        out_type=jax.ShapeDtypeStruct((2, NPAD * D), jnp.float32),
        scratch_types=[
            pltpu.VMEM((SCC,), jnp.int32),
            pltpu.VMEM((SCC,), jnp.int32),
            pltpu.VMEM((EC, D), jnp.float32),
            pltpu.VMEM((EC, D), jnp.float32),
            pltpu.VMEM((RPW * D + D,), jnp.float32),
            pltpu.VMEM((NSUB,), jnp.int32),
            pltpu.SemaphoreType.DMA,
            pltpu.SemaphoreType.DMA,
        ],
    )(_seg_body)
    return deg, seg


def kernel(user_emb, item_emb, v_feat, t_feat, W_v, b_v, W_t, b_t, edge_index):
    fnv, fnt, hv, ht = _prep(v_feat, t_feat, item_emb, W_v, b_v, W_t, b_t)
    v_out = _knn(fnv, fnv.T, hv, 512)
    t_out = _knn(fnt, fnt.T, ht, 384)
    item_h = item_emb + v_out + t_out

    uids = edge_index[0].astype(jnp.int32)
    iids = edge_index[1].astype(jnp.int32)
    keyi = iids * PACK + uids
    keyu = uids * PACK + iids
    keyi = lax.sort(keyi)
    keyu = lax.sort(keyu)
    zpad = jnp.zeros((EPAD - E,), jnp.int32)
    keyi_p = jnp.concatenate([keyi, zpad])
    keyu_p = jnp.concatenate([keyu, zpad])
    marks = jnp.arange(NSUB + 1, dtype=jnp.int32) * (RPW * PACK)
    bnd_i = jnp.searchsorted(keyi, marks).astype(jnp.int32)
    bnd_u = jnp.searchsorted(keyu, marks).astype(jnp.int32)
    lo = jnp.stack([bnd_i[:NSUB], bnd_u[:NSUB]])
    hi = jnp.stack([bnd_i[1:], bnd_u[1:]])

    deg_kernel, seg_kernel = _sc_kernels()
    cnt = deg_kernel(keyi_p, keyu_p, lo, hi)
    deg_i = cnt[0].reshape(NPAD, 16)[:NI, 0]
    deg_u = cnt[1].reshape(NPAD, 16)[:NU, 0]
    a = 1.0 / jnp.sqrt(jnp.maximum(deg_u, 1.0))
    b = 1.0 / jnp.sqrt(jnp.maximum(deg_i, 1.0))

    s1 = seg_kernel(user_emb * a[:, None], item_h * b[:, None],
                    keyi_p, keyu_p, lo, hi)
    xi1 = s1[0].reshape(NPAD, D)[:NI] * b[:, None]
    xu1 = s1[1].reshape(NPAD, D)[:NU] * a[:, None]

    s2 = seg_kernel(xu1 * a[:, None], xi1 * b[:, None],
                    keyi_p, keyu_p, lo, hi)
    xi2 = s2[0].reshape(NPAD, D)[:NI] * b[:, None]
    xu2 = s2[1].reshape(NPAD, D)[:NU] * a[:, None]

    emb_u = (user_emb + xu1 + xu2) * (1.0 / 3.0)
    emb_i = (item_h + xi1 + xi2) * (1.0 / 3.0)
    return jnp.concatenate([emb_u, emb_i], axis=0)


# speedup vs baseline: 5.0579x; 1.0030x over previous
import dataclasses
import functools

import jax
import jax.numpy as jnp
from jax import lax
from jax.experimental import pallas as pl
from jax.experimental.pallas import tpu as pltpu
from jax.experimental.pallas import tpu_sc as plsc

NU = 5000
NI = 5000
D = 256
KNN = 10
E = 160000

RB = 200
NB = NI // RB
KRB = 128
KNR = 5120
KNB = KNR // KRB

NSUB = 16
NPAD = 5120
RPW = NPAD // NSUB
EC = 32
SCC = 1024
NCPS = SCC // EC
EPAD = -(-E // SCC) * SCC
ECD = 128
PACK = 8192
SHIFT = 13


def _prep_body(vf_ref, tf_ref, ie_ref, wv_ref, bv_ref, wt_ref, bt_ref,
               fnv_ref, fnt_ref, hv_ref, ht_ref):
    vf = vf_ref[...]
    tf = tf_ref[...]
    ie = ie_ref[...]
    nv = jnp.sqrt(jnp.sum(vf * vf, axis=1, keepdims=True)) + 1e-8
    fnv_ref[...] = vf / nv
    nt = jnp.sqrt(jnp.sum(tf * tf, axis=1, keepdims=True)) + 1e-8
    fnt_ref[...] = tf / nt
    hv_ref[...] = ie + jnp.dot(vf, wv_ref[...],
                               preferred_element_type=jnp.float32) + bv_ref[...]
    ht_ref[...] = ie + jnp.dot(tf, wt_ref[...],
                               preferred_element_type=jnp.float32) + bt_ref[...]


def _prep(v_feat, t_feat, item_emb, W_v, b_v, W_t, b_t):
    return pl.pallas_call(
        _prep_body,
        grid=(NB,),
        in_specs=[
            pl.BlockSpec((RB, 512), lambda i: (i, 0)),
            pl.BlockSpec((RB, 384), lambda i: (i, 0)),
            pl.BlockSpec((RB, D), lambda i: (i, 0)),
            pl.BlockSpec((512, D), lambda i: (0, 0)),
            pl.BlockSpec((1, D), lambda i: (0, 0)),
            pl.BlockSpec((384, D), lambda i: (0, 0)),
            pl.BlockSpec((1, D), lambda i: (0, 0)),
        ],
        out_specs=[
            pl.BlockSpec((RB, 512), lambda i: (i, 0)),
            pl.BlockSpec((RB, 384), lambda i: (i, 0)),
            pl.BlockSpec((RB, D), lambda i: (i, 0)),
            pl.BlockSpec((RB, D), lambda i: (i, 0)),
        ],
        out_shape=[
            jax.ShapeDtypeStruct((NI, 512), jnp.float32),
            jax.ShapeDtypeStruct((NI, 384), jnp.float32),
            jax.ShapeDtypeStruct((NI, D), jnp.float32),
            jax.ShapeDtypeStruct((NI, D), jnp.float32),
        ],
    )(v_feat, t_feat, item_emb, W_v, b_v.reshape(1, D), W_t, b_t.reshape(1, D))


def _knn_body(fb_ref, fT_ref, h_ref, o_ref):
    fb = fb_ref[...]
    scores = jnp.dot(fb, fT_ref[...], preferred_element_type=jnp.float32)
    m0 = jnp.max(scores, axis=1, keepdims=True)
    sel = jnp.zeros(scores.shape, jnp.float32)
    denom = jnp.zeros((scores.shape[0], 1), jnp.float32)
    for t in range(KNN):
        m = m0 if t == 0 else jnp.max(scores, axis=1, keepdims=True)
        w = jnp.exp(m - m0)
        hit = scores >= m
        sel = sel + w * hit.astype(jnp.float32)
        denom = denom + w
        scores = jnp.where(hit, -1e30, scores)
    out = jnp.dot(sel, h_ref[...], preferred_element_type=jnp.float32)
    o_ref[...] = out / denom


def _knn(fn, fnT, h, kd):
    fn_pad = jnp.concatenate(
        [fn, jnp.zeros((KNR - NI, kd), jnp.float32)], axis=0)
    out = pl.pallas_call(
        _knn_body,
        grid=(KNB,),
        in_specs=[
            pl.BlockSpec((KRB, kd), lambda i: (i, 0)),
            pl.BlockSpec((kd, NI), lambda i: (0, 0)),
            pl.BlockSpec((NI, D), lambda i: (0, 0)),
        ],
        out_specs=pl.BlockSpec((KRB, D), lambda i: (i, 0)),
        out_shape=jax.ShapeDtypeStruct((KNR, D), jnp.float32),
    )(fn_pad, fnT, h)
    return out[:NI]


def _worker_bounds(lo_hbm, hi_hbm, bnd_v, cid, sid):
    iota = lax.iota(jnp.int32, NSUB)
    pltpu.sync_copy(lo_hbm.at[cid], bnd_v)
    lo = jnp.sum(jnp.where(iota == sid, bnd_v[...], 0))
    pltpu.sync_copy(hi_hbm.at[cid], bnd_v)
    hi = jnp.sum(jnp.where(iota == sid, bnd_v[...], 0))
    return lo, hi


def _deg_body(keyi_hbm, keyu_hbm, lo_hbm, hi_hbm, out_hbm,
              kidx_v, acc_v, bnd_v, sem):
    cid = lax.axis_index("c")
    sid = lax.axis_index("s")
    lo, hi = _worker_bounds(lo_hbm, hi_hbm, bnd_v, cid, sid)
    rbase = sid * RPW
    iota = lax.iota(jnp.int32, 16)

    @pl.loop(0, RPW + 1)
    def _(i):
        acc_v[pl.ds(i * 16, 16)] = jnp.zeros((16,), jnp.float32)

    ones = jnp.ones((16,), jnp.float32)

    def scan(key_hbm):
        @pl.loop(lo // ECD, (hi + ECD - 1) // ECD)
        def _(t):
            base = t * ECD
            pltpu.sync_copy(key_hbm.at[pl.ds(base, ECD)], kidx_v)
            for jv in range(ECD // 16):
                kvec = kidx_v[pl.ds(jv * 16, 16)]
                dvec = lax.shift_right_logical(kvec, SHIFT)
                gvec = base + jv * 16 + iota
                valid = jnp.logical_and(gvec >= lo, gvec < hi)
                offv = jnp.where(valid, dvec - rbase, RPW) * 16
                for j in range(16):
                    plsc.addupdate(acc_v.at[pl.ds(offv[j], 16)], ones)

    @pl.when(cid == 0)
    def _():
        scan(keyi_hbm)

    @pl.when(cid == 1)
    def _():
        scan(keyu_hbm)

    pltpu.sync_copy(acc_v.at[pl.ds(0, RPW * 16)],
                    out_hbm.at[cid].at[pl.ds(sid * RPW * 16, RPW * 16)])


def _seg_body(yu_hbm, yi_hbm, keyi_hbm, keyu_hbm,
              lo_hbm, hi_hbm, out_hbm, sidx_v, kidx_v, rows0_v, rows1_v,
              acc_v, bnd_v, sem0, sem1):
    cid = lax.axis_index("c")
    sid = lax.axis_index("s")
    lo, hi = _worker_bounds(lo_hbm, hi_hbm, bnd_v, cid, sid)
    rbase = sid * RPW
    iota = lax.iota(jnp.int32, 16)

    @pl.loop(0, RPW * (D // 16) + D // 16)
    def _(i):
        acc_v[pl.ds(i * 16, 16)] = jnp.zeros((16,), jnp.float32)

    def scan(tbl_hbm, key_hbm):
        rows = (rows0_v, rows1_v)
        sems = (sem0, sem1)

        def issue(k, b):
            pltpu.async_copy(tbl_hbm.at[sidx_v.at[pl.ds(k * EC, EC)]],
                             rows[b], sems[b])

        def process(k, b, base):
            for jv in range(EC // 16):
                kvec = kidx_v[pl.ds(k * EC + jv * 16, 16)]
                dvec = lax.shift_right_logical(kvec, SHIFT)
                gvec = base + k * EC + jv * 16 + iota
                valid = jnp.logical_and(gvec >= lo, gvec < hi)
                offv = jnp.where(valid, dvec - rbase, RPW) * D
                offs = [offv[j] for j in range(16)]
                for j in range(16):
                    for kk in range(D // 16):
                        val = rows[b][jv * 16 + j, pl.ds(kk * 16, 16)]
                        plsc.addupdate(
                            acc_v.at[pl.ds(offs[j] + kk * 16, 16)], val)

        @pl.loop(lo // SCC, (hi + SCC - 1) // SCC)
        def _(sb):
            base = sb * SCC
            pltpu.sync_copy(key_hbm.at[pl.ds(base, SCC)], kidx_v)

            @pl.loop(0, SCC // 16)
            def _(i):
                sidx_v[pl.ds(i * 16, 16)] = jnp.bitwise_and(
                    kidx_v[pl.ds(i * 16, 16)], PACK - 1)

            issue(0, 0)
            issue(1, 1)

            @pl.loop(0, NCPS, step=2)
            def _(k):
                pltpu.make_async_copy(
                    tbl_hbm.at[sidx_v.at[pl.ds(0, EC)]], rows[0],
                    sems[0]).wait()
                process(k, 0, base)

                @pl.when(k + 2 < NCPS)
                def _():
                    issue(k + 2, 0)

                pltpu.make_async_copy(
                    tbl_hbm.at[sidx_v.at[pl.ds(0, EC)]], rows[1],
                    sems[1]).wait()
                process(k + 1, 1, base)

                @pl.when(k + 3 < NCPS)
                def _():
                    issue(k + 3, 1)

    @pl.when(cid == 0)
    def _():
        scan(yu_hbm, keyi_hbm)

    @pl.when(cid == 1)
    def _():
        scan(yi_hbm, keyu_hbm)

    pltpu.sync_copy(acc_v.at[pl.ds(0, RPW * D)],
                    out_hbm.at[cid].at[pl.ds(sid * RPW * D, RPW * D)])


@functools.cache
def _sc_kernels():
    mesh = plsc.VectorSubcoreMesh(core_axis_name="c", subcore_axis_name="s")
    cp = pltpu.CompilerParams()
    if "needs_layout_passes" in pltpu.CompilerParams.__dataclass_fields__:
        cp = dataclasses.replace(cp, needs_layout_passes=False)

    deg = functools.partial(
        pl.kernel, mesh=mesh, compiler_params=cp,
        out_type=jax.ShapeDtypeStruct((2, NPAD * 16), jnp.float32),
        scratch_types=[
            pltpu.VMEM((ECD,), jnp.int32),
            pltpu.VMEM((RPW * 16 + 16,), jnp.float32),
            pltpu.VMEM((NSUB,), jnp.int32),
            pltpu.SemaphoreType.DMA,
        ],
    )(_deg_body)
    seg = functools.partial(
        pl.kernel, mesh=mesh, compiler_params=cp,
        out_type=jax.ShapeDtypeStruct((2, NPAD * D), jnp.float32),
        scratch_types=[
            pltpu.VMEM((SCC,), jnp.int32),
            pltpu.VMEM((SCC,), jnp.int32),
            pltpu.VMEM((EC, D), jnp.float32),
            pltpu.VMEM((EC, D), jnp.float32),
            pltpu.VMEM((RPW * D + D,), jnp.float32),
            pltpu.VMEM((NSUB,), jnp.int32),
            pltpu.SemaphoreType.DMA,
            pltpu.SemaphoreType.DMA,
        ],
    )(_seg_body)
    return deg, seg


def kernel(user_emb, item_emb, v_feat, t_feat, W_v, b_v, W_t, b_t, edge_index):
    fnv, fnt, hv, ht = _prep(v_feat, t_feat, item_emb, W_v, b_v, W_t, b_t)
    v_out = _knn(fnv, fnv.T, hv, 512)
    t_out = _knn(fnt, fnt.T, ht, 384)
    item_h = item_emb + v_out + t_out

    uids = edge_index[0].astype(jnp.int32)
    iids = edge_index[1].astype(jnp.int32)
    keyi = iids * PACK + uids
    keyu = uids * PACK + iids
    keyi = lax.sort(keyi)
    keyu = lax.sort(keyu)
    zpad = jnp.zeros((EPAD - E,), jnp.int32)
    keyi_p = jnp.concatenate([keyi, zpad])
    keyu_p = jnp.concatenate([keyu, zpad])
    marks = jnp.arange(NSUB + 1, dtype=jnp.int32) * (RPW * PACK)
    bnd_i = jnp.searchsorted(keyi, marks).astype(jnp.int32)
    bnd_u = jnp.searchsorted(keyu, marks).astype(jnp.int32)
    lo = jnp.stack([bnd_i[:NSUB], bnd_u[:NSUB]])
    hi = jnp.stack([bnd_i[1:], bnd_u[1:]])

    deg_kernel, seg_kernel = _sc_kernels()
    cnt = deg_kernel(keyi_p, keyu_p, lo, hi)
    deg_i = cnt[0].reshape(NPAD, 16)[:NI, 0]
    deg_u = cnt[1].reshape(NPAD, 16)[:NU, 0]
    a = 1.0 / jnp.sqrt(jnp.maximum(deg_u, 1.0))
    b = 1.0 / jnp.sqrt(jnp.maximum(deg_i, 1.0))

    s1 = seg_kernel(user_emb * a[:, None], item_h * b[:, None],
                    keyi_p, keyu_p, lo, hi)
    xi1 = s1[0].reshape(NPAD, D)[:NI] * b[:, None]
    xu1 = s1[1].reshape(NPAD, D)[:NU] * a[:, None]

    s2 = seg_kernel(xu1 * a[:, None], xi1 * b[:, None],
                    keyi_p, keyu_p, lo, hi)
    xi2 = s2[0].reshape(NPAD, D)[:NI] * b[:, None]
    xu2 = s2[1].reshape(NPAD, D)[:NU] * a[:, None]

    emb_u = (user_emb + xu1 + xu2) * (1.0 / 3.0)
    emb_i = (item_h + xi1 + xi2) * (1.0 / 3.0)
    return jnp.concatenate([emb_u, emb_i], axis=0)


# speedup vs baseline: 5.7065x; 1.1282x over previous
import dataclasses
import functools

import jax
import jax.numpy as jnp
from jax import lax
from jax.experimental import pallas as pl
from jax.experimental.pallas import tpu as pltpu
from jax.experimental.pallas import tpu_sc as plsc

NU = 5000
NI = 5000
D = 256
KNN = 10
E = 160000

RB = 200
NB = NI // RB
KRB = 128
KNR = 5120
KNB = KNR // KRB

NSUB = 16
NPAD = 5120
RPW = NPAD // NSUB
EC = 32
SCC = 1024
NCPS = SCC // EC
EPAD = -(-E // SCC) * SCC
ECD = 128
PACK = 8192
SHIFT = 13


def _prep_body(vf_ref, tf_ref, ie_ref, wv_ref, bv_ref, wt_ref, bt_ref,
               fnv_ref, fnt_ref, hv_ref, ht_ref):
    vf = vf_ref[...]
    tf = tf_ref[...]
    ie = ie_ref[...]
    nv = jnp.sqrt(jnp.sum(vf * vf, axis=1, keepdims=True)) + 1e-8
    fnv_ref[...] = vf / nv
    nt = jnp.sqrt(jnp.sum(tf * tf, axis=1, keepdims=True)) + 1e-8
    fnt_ref[...] = tf / nt
    hv_ref[...] = ie + jnp.dot(vf, wv_ref[...],
                               preferred_element_type=jnp.float32) + bv_ref[...]
    ht_ref[...] = ie + jnp.dot(tf, wt_ref[...],
                               preferred_element_type=jnp.float32) + bt_ref[...]


def _prep(v_feat, t_feat, item_emb, W_v, b_v, W_t, b_t):
    return pl.pallas_call(
        _prep_body,
        grid=(NB,),
        in_specs=[
            pl.BlockSpec((RB, 512), lambda i: (i, 0)),
            pl.BlockSpec((RB, 384), lambda i: (i, 0)),
            pl.BlockSpec((RB, D), lambda i: (i, 0)),
            pl.BlockSpec((512, D), lambda i: (0, 0)),
            pl.BlockSpec((1, D), lambda i: (0, 0)),
            pl.BlockSpec((384, D), lambda i: (0, 0)),
            pl.BlockSpec((1, D), lambda i: (0, 0)),
        ],
        out_specs=[
            pl.BlockSpec((RB, 512), lambda i: (i, 0)),
            pl.BlockSpec((RB, 384), lambda i: (i, 0)),
            pl.BlockSpec((RB, D), lambda i: (i, 0)),
            pl.BlockSpec((RB, D), lambda i: (i, 0)),
        ],
        out_shape=[
            jax.ShapeDtypeStruct((NI, 512), jnp.float32),
            jax.ShapeDtypeStruct((NI, 384), jnp.float32),
            jax.ShapeDtypeStruct((NI, D), jnp.float32),
            jax.ShapeDtypeStruct((NI, D), jnp.float32),
        ],
    )(v_feat, t_feat, item_emb, W_v, b_v.reshape(1, D), W_t, b_t.reshape(1, D))


def _knn_body(fb_ref, fT_ref, h_ref, o_ref):
    fb = fb_ref[...]
    scores = jnp.dot(fb, fT_ref[...], preferred_element_type=jnp.float32)
    m0 = jnp.max(scores, axis=1, keepdims=True)
    sel = jnp.zeros(scores.shape, jnp.float32)
    denom = jnp.zeros((scores.shape[0], 1), jnp.float32)
    for t in range(KNN):
        m = m0 if t == 0 else jnp.max(scores, axis=1, keepdims=True)
        w = jnp.exp(m - m0)
        hit = scores >= m
        sel = sel + w * hit.astype(jnp.float32)
        denom = denom + w
        scores = jnp.where(hit, -1e30, scores)
    out = jnp.dot(sel, h_ref[...], preferred_element_type=jnp.float32)
    o_ref[...] = out / denom


def _knn(fn, fnT, h, kd):
    fn_pad = jnp.concatenate(
        [fn, jnp.zeros((KNR - NI, kd), jnp.float32)], axis=0)
    out = pl.pallas_call(
        _knn_body,
        grid=(KNB,),
        in_specs=[
            pl.BlockSpec((KRB, kd), lambda i: (i, 0)),
            pl.BlockSpec((kd, NI), lambda i: (0, 0)),
            pl.BlockSpec((NI, D), lambda i: (0, 0)),
        ],
        out_specs=pl.BlockSpec((KRB, D), lambda i: (i, 0)),
        out_shape=jax.ShapeDtypeStruct((KNR, D), jnp.float32),
    )(fn_pad, fnT, h)
    return out[:NI]


def _worker_bounds(lo_hbm, hi_hbm, bnd_v, cid, sid):
    iota = lax.iota(jnp.int32, NSUB)
    pltpu.sync_copy(lo_hbm.at[cid], bnd_v)
    lo = jnp.sum(jnp.where(iota == sid, bnd_v[...], 0))
    pltpu.sync_copy(hi_hbm.at[cid], bnd_v)
    hi = jnp.sum(jnp.where(iota == sid, bnd_v[...], 0))
    return lo, hi


def _deg_body(keyi_hbm, keyu_hbm, lo_hbm, hi_hbm, out_hbm,
              kidx_v, acc_v, bnd_v, sem):
    cid = lax.axis_index("c")
    sid = lax.axis_index("s")
    lo, hi = _worker_bounds(lo_hbm, hi_hbm, bnd_v, cid, sid)
    rbase = sid * RPW
    iota = lax.iota(jnp.int32, 16)

    @pl.loop(0, RPW + 1)
    def _(i):
        acc_v[pl.ds(i * 16, 16)] = jnp.zeros((16,), jnp.float32)

    ones = jnp.ones((16,), jnp.float32)

    def scan(key_hbm):
        @pl.loop(lo // ECD, (hi + ECD - 1) // ECD)
        def _(t):
            base = t * ECD
            pltpu.sync_copy(key_hbm.at[pl.ds(base, ECD)], kidx_v)
            for jv in range(ECD // 16):
                kvec = kidx_v[pl.ds(jv * 16, 16)]
                dvec = lax.shift_right_logical(kvec, SHIFT)
                gvec = base + jv * 16 + iota
                valid = jnp.logical_and(gvec >= lo, gvec < hi)
                offv = jnp.where(valid, dvec - rbase, RPW) * 16
                for j in range(16):
                    plsc.addupdate(acc_v.at[pl.ds(offv[j], 16)], ones)

    @pl.when(cid == 0)
    def _():
        scan(keyi_hbm)

    @pl.when(cid == 1)
    def _():
        scan(keyu_hbm)

    pltpu.sync_copy(acc_v.at[pl.ds(0, RPW * 16)],
                    out_hbm.at[cid].at[pl.ds(sid * RPW * 16, RPW * 16)])


def _seg_body(yu_hbm, yi_hbm, keyi_hbm, keyu_hbm,
              lo_hbm, hi_hbm, out_hbm, sidx_v, kidx_v, rows0_v, rows1_v,
              acc_v, bnd_v, sem0, sem1):
    cid = lax.axis_index("c")
    sid = lax.axis_index("s")
    lo, hi = _worker_bounds(lo_hbm, hi_hbm, bnd_v, cid, sid)
    rbase = sid * RPW
    iota = lax.iota(jnp.int32, 16)

    @pl.loop(0, RPW * (D // 16) + D // 16)
    def _(i):
        acc_v[pl.ds(i * 16, 16)] = jnp.zeros((16,), jnp.float32)

    def scan(tbl_hbm, key_hbm):
        rows = (rows0_v, rows1_v)
        sems = (sem0, sem1)

        def issue(k, b):
            pltpu.async_copy(tbl_hbm.at[sidx_v.at[pl.ds(k * EC, EC)]],
                             rows[b], sems[b])

        def process(k, b, base):
            for jv in range(EC // 16):
                kvec = kidx_v[pl.ds(k * EC + jv * 16, 16)]
                dvec = lax.shift_right_logical(kvec, SHIFT)
                gvec = base + k * EC + jv * 16 + iota
                valid = jnp.logical_and(gvec >= lo, gvec < hi)
                offv = jnp.where(valid, dvec - rbase, RPW) * D
                offs = [offv[j] for j in range(16)]
                for j in range(16):
                    for kk in range(D // 16):
                        val = rows[b][jv * 16 + j, pl.ds(kk * 16, 16)]
                        plsc.addupdate(
                            acc_v.at[pl.ds(offs[j] + kk * 16, 16)], val)

        @pl.loop(lo // SCC, (hi + SCC - 1) // SCC)
        def _(sb):
            base = sb * SCC
            pltpu.sync_copy(key_hbm.at[pl.ds(base, SCC)], kidx_v)

            @pl.loop(0, SCC // 16)
            def _(i):
                sidx_v[pl.ds(i * 16, 16)] = jnp.bitwise_and(
                    kidx_v[pl.ds(i * 16, 16)], PACK - 1)

            issue(0, 0)
            issue(1, 1)

            @pl.loop(0, NCPS, step=2)
            def _(k):
                pltpu.make_async_copy(
                    tbl_hbm.at[sidx_v.at[pl.ds(0, EC)]], rows[0],
                    sems[0]).wait()
                process(k, 0, base)

                @pl.when(k + 2 < NCPS)
                def _():
                    issue(k + 2, 0)

                pltpu.make_async_copy(
                    tbl_hbm.at[sidx_v.at[pl.ds(0, EC)]], rows[1],
                    sems[1]).wait()
                process(k + 1, 1, base)

                @pl.when(k + 3 < NCPS)
                def _():
                    issue(k + 3, 1)

    @pl.when(cid == 0)
    def _():
        scan(yu_hbm, keyi_hbm)

    @pl.when(cid == 1)
    def _():
        scan(yi_hbm, keyu_hbm)

    pltpu.sync_copy(acc_v.at[pl.ds(0, RPW * D)],
                    out_hbm.at[cid].at[pl.ds(sid * RPW * D, RPW * D)])


RPW1 = NPAD // 32


def _seg1_body(tbl_hbm, key_hbm, lo_hbm, hi_hbm, out_hbm,
               sidx_v, kidx_v, rows0_v, rows1_v, acc_v, bnd_v, sem0, sem1):
    cid = lax.axis_index("c")
    sid = lax.axis_index("s")
    lo, hi = _worker_bounds(lo_hbm, hi_hbm, bnd_v, cid, sid)
    wid = cid * NSUB + sid
    rbase = wid * RPW1
    iota = lax.iota(jnp.int32, 16)

    @pl.loop(0, RPW1 * (D // 16) + D // 16)
    def _(i):
        acc_v[pl.ds(i * 16, 16)] = jnp.zeros((16,), jnp.float32)

    rows = (rows0_v, rows1_v)
    sems = (sem0, sem1)

    def issue(k, b):
        pltpu.async_copy(tbl_hbm.at[sidx_v.at[pl.ds(k * EC, EC)]],
                         rows[b], sems[b])

    def process(k, b, base):
        for jv in range(EC // 16):
            kvec = kidx_v[pl.ds(k * EC + jv * 16, 16)]
            dvec = lax.shift_right_logical(kvec, SHIFT)
            gvec = base + k * EC + jv * 16 + iota
            valid = jnp.logical_and(gvec >= lo, gvec < hi)
            offv = jnp.where(valid, dvec - rbase, RPW1) * D
            offs = [offv[j] for j in range(16)]
            for j in range(16):
                for kk in range(D // 16):
                    val = rows[b][jv * 16 + j, pl.ds(kk * 16, 16)]
                    plsc.addupdate(
                        acc_v.at[pl.ds(offs[j] + kk * 16, 16)], val)

    @pl.loop(lo // SCC, (hi + SCC - 1) // SCC)
    def _(sb):
        base = sb * SCC
        pltpu.sync_copy(key_hbm.at[pl.ds(base, SCC)], kidx_v)

        @pl.loop(0, SCC // 16)
        def _(i):
            sidx_v[pl.ds(i * 16, 16)] = jnp.bitwise_and(
                kidx_v[pl.ds(i * 16, 16)], PACK - 1)

        issue(0, 0)
        issue(1, 1)

        @pl.loop(0, NCPS, step=2)
        def _(k):
            pltpu.make_async_copy(
                tbl_hbm.at[sidx_v.at[pl.ds(0, EC)]], rows[0], sems[0]).wait()
            process(k, 0, base)

            @pl.when(k + 2 < NCPS)
            def _():
                issue(k + 2, 0)

            pltpu.make_async_copy(
                tbl_hbm.at[sidx_v.at[pl.ds(0, EC)]], rows[1], sems[1]).wait()
            process(k + 1, 1, base)

            @pl.when(k + 3 < NCPS)
            def _():
                issue(k + 3, 1)

    pltpu.sync_copy(acc_v.at[pl.ds(0, RPW1 * D)],
                    out_hbm.at[pl.ds(wid * RPW1 * D, RPW1 * D)])


@functools.cache
def _sc_kernels():
    mesh = plsc.VectorSubcoreMesh(core_axis_name="c", subcore_axis_name="s")
    cp = pltpu.CompilerParams()
    if "needs_layout_passes" in pltpu.CompilerParams.__dataclass_fields__:
        cp = dataclasses.replace(cp, needs_layout_passes=False)

    deg = functools.partial(
        pl.kernel, mesh=mesh, compiler_params=cp,
        out_type=jax.ShapeDtypeStruct((2, NPAD * 16), jnp.float32),
        scratch_types=[
            pltpu.VMEM((ECD,), jnp.int32),
            pltpu.VMEM((RPW * 16 + 16,), jnp.float32),
            pltpu.VMEM((NSUB,), jnp.int32),
            pltpu.SemaphoreType.DMA,
        ],
    )(_deg_body)
    seg = functools.partial(
        pl.kernel, mesh=mesh, compiler_params=cp,
        out_type=jax.ShapeDtypeStruct((2, NPAD * D), jnp.float32),
        scratch_types=[
            pltpu.VMEM((SCC,), jnp.int32),
            pltpu.VMEM((SCC,), jnp.int32),
            pltpu.VMEM((EC, D), jnp.float32),
            pltpu.VMEM((EC, D), jnp.float32),
            pltpu.VMEM((RPW * D + D,), jnp.float32),
            pltpu.VMEM((NSUB,), jnp.int32),
            pltpu.SemaphoreType.DMA,
            pltpu.SemaphoreType.DMA,
        ],
    )(_seg_body)
    seg1 = functools.partial(
        pl.kernel, mesh=mesh, compiler_params=cp,
        out_type=jax.ShapeDtypeStruct((NPAD * D,), jnp.float32),
        scratch_types=[
            pltpu.VMEM((SCC,), jnp.int32),
            pltpu.VMEM((SCC,), jnp.int32),
            pltpu.VMEM((EC, D), jnp.float32),
            pltpu.VMEM((EC, D), jnp.float32),
            pltpu.VMEM((RPW1 * D + D,), jnp.float32),
            pltpu.VMEM((NSUB,), jnp.int32),
            pltpu.SemaphoreType.DMA,
            pltpu.SemaphoreType.DMA,
        ],
    )(_seg1_body)
    return deg, seg, seg1


def kernel(user_emb, item_emb, v_feat, t_feat, W_v, b_v, W_t, b_t, edge_index):
    uids = edge_index[0].astype(jnp.int32)
    iids = edge_index[1].astype(jnp.int32)
    keyi = iids * PACK + uids
    keyu = uids * PACK + iids
    keyi = lax.sort(keyi)
    keyu = lax.sort(keyu)
    zpad = jnp.zeros((EPAD - E,), jnp.int32)
    keyi_p = jnp.concatenate([keyi, zpad])
    keyu_p = jnp.concatenate([keyu, zpad])
    marks = jnp.arange(NSUB + 1, dtype=jnp.int32) * (RPW * PACK)
    bnd_i = jnp.searchsorted(keyi, marks).astype(jnp.int32)
    bnd_u = jnp.searchsorted(keyu, marks).astype(jnp.int32)
    lo = jnp.stack([bnd_i[:NSUB], bnd_u[:NSUB]])
    hi = jnp.stack([bnd_i[1:], bnd_u[1:]])
    marks1 = jnp.arange(2 * NSUB + 1, dtype=jnp.int32) * (RPW1 * PACK)
    b1i = jnp.searchsorted(keyi, marks1).astype(jnp.int32)
    b1u = jnp.searchsorted(keyu, marks1).astype(jnp.int32)

    deg_kernel, seg_kernel, seg1_kernel = _sc_kernels()
    cnt = deg_kernel(keyi_p, keyu_p, lo, hi)
    deg_i = cnt[0].reshape(NPAD, 16)[:NI, 0]
    deg_u = cnt[1].reshape(NPAD, 16)[:NU, 0]
    a = 1.0 / jnp.sqrt(jnp.maximum(deg_u, 1.0))
    b = 1.0 / jnp.sqrt(jnp.maximum(deg_i, 1.0))

    s1a = seg1_kernel(user_emb * a[:, None], keyi_p,
                      b1i[:32].reshape(2, NSUB), b1i[1:].reshape(2, NSUB))

    fnv, fnt, hv, ht = _prep(v_feat, t_feat, item_emb, W_v, b_v, W_t, b_t)
    v_out = _knn(fnv, fnv.T, hv, 512)
    t_out = _knn(fnt, fnt.T, ht, 384)
    item_h = item_emb + v_out + t_out

    s1b = seg1_kernel(item_h * b[:, None], keyu_p,
                      b1u[:32].reshape(2, NSUB), b1u[1:].reshape(2, NSUB))
    xi1 = s1a.reshape(NPAD, D)[:NI] * b[:, None]
    xu1 = s1b.reshape(NPAD, D)[:NU] * a[:, None]

    s2 = seg_kernel(xu1 * a[:, None], xi1 * b[:, None],
                    keyi_p, keyu_p, lo, hi)
    xi2 = s2[0].reshape(NPAD, D)[:NI] * b[:, None]
    xu2 = s2[1].reshape(NPAD, D)[:NU] * a[:, None]

    emb_u = (user_emb + xu1 + xu2) * (1.0 / 3.0)
    emb_i = (item_h + xi1 + xi2) * (1.0 / 3.0)
    return jnp.concatenate([emb_u, emb_i], axis=0)
